# baseline (speedup 1.0000x reference)
import jax
import jax.numpy as jnp
from jax import lax
from jax.experimental import pallas as pl
from jax.experimental.pallas import tpu as pltpu


def _chamfer_body(src_ref, tgt_ref, out_ref):
    b = pl.program_id(0)
    nb = pl.num_programs(0)
    src = src_ref[0]
    tgt = tgt_ref[0]

    sq_s = jnp.sum(src * src, axis=1, keepdims=True)
    sq_t = jnp.sum(tgt * tgt, axis=1, keepdims=True)

    lane_s = lax.broadcasted_iota(jnp.int32, src.shape, 1)
    lane_t = lax.broadcasted_iota(jnp.int32, tgt.shape, 1)

    src_aug = jnp.where(
        lane_s < 3, -2.0 * src,
        jnp.where(lane_s == 3, 1.0, jnp.where(lane_s == 4, sq_s, 0.0)),
    )
    tgt_aug = jnp.where(
        lane_t < 3, tgt,
        jnp.where(lane_t == 3, sq_t, jnp.where(lane_t == 4, 1.0, 0.0)),
    )
    d2 = lax.dot_general(
        src_aug, tgt_aug, (((1,), (1,)), ((), ())),
        preferred_element_type=jnp.float32,
    )

    rowmin = jnp.min(d2, axis=1, keepdims=True)
    colmin = jnp.min(d2, axis=0, keepdims=True)

    n = src.shape[0]
    m = tgt.shape[0]
    batch_val = (
        jnp.sum(jnp.maximum(rowmin, 0.0)) / n
        + jnp.sum(jnp.maximum(colmin, 0.0)) / m
    )

    @pl.when(b == 0)
    def _():
        out_ref[0, 0] = 0.0

    out_ref[0, 0] += batch_val / nb


@jax.jit
def kernel(src_points, tgt_points):
    B, N, D = src_points.shape
    M = tgt_points.shape[1]
    src_pad = jnp.pad(src_points, ((0, 0), (0, 0), (0, 8 - D)))
    tgt_pad = jnp.pad(tgt_points, ((0, 0), (0, 0), (0, 8 - D)))

    out = pl.pallas_call(
        _chamfer_body,
        grid=(B,),
        in_specs=[
            pl.BlockSpec((1, N, 8), lambda b: (b, 0, 0)),
            pl.BlockSpec((1, M, 8), lambda b: (b, 0, 0)),
        ],
        out_specs=pl.BlockSpec((1, 1), lambda b: (0, 0), memory_space=pltpu.SMEM),
        out_shape=jax.ShapeDtypeStruct((1, 1), jnp.float32),
    )(src_pad, tgt_pad)
    return out[0, 0]


# speedup vs baseline: 1.4938x; 1.4938x over previous
import jax
import jax.numpy as jnp
from jax import lax
from jax.experimental import pallas as pl
from jax.experimental.pallas import tpu as pltpu


def _chamfer_body(src_ref, tgtT_ref, out_ref):
    b = pl.program_id(0)
    nb = pl.num_programs(0)
    src_aug = src_ref[0]
    tgt_aug = tgtT_ref[0]

    d2 = jnp.dot(src_aug, tgt_aug, preferred_element_type=jnp.float32)

    rowmin = jnp.min(d2, axis=1, keepdims=True)
    colmin = jnp.min(d2, axis=0, keepdims=True)

    n = src_aug.shape[0]
    m = tgt_aug.shape[1]
    batch_val = (
        jnp.sum(jnp.maximum(rowmin, 0.0)) / n
        + jnp.sum(jnp.maximum(colmin, 0.0)) / m
    )

    @pl.when(b == 0)
    def _():
        out_ref[0, 0] = 0.0

    out_ref[0, 0] += batch_val / nb


@jax.jit
def kernel(src_points, tgt_points):
    B, N, D = src_points.shape
    M = tgt_points.shape[1]

    sq_s = jnp.sum(src_points * src_points, axis=-1, keepdims=True)
    sq_t = jnp.sum(tgt_points * tgt_points, axis=-1, keepdims=True)
    ones_s = jnp.ones((B, N, 1), jnp.float32)
    ones_t = jnp.ones((B, M, 1), jnp.float32)
    src_aug = jnp.concatenate(
        [-2.0 * src_points, ones_s, sq_s, jnp.zeros((B, N, 3), jnp.float32)], axis=-1
    )
    tgtT_aug = jnp.transpose(
        jnp.concatenate(
            [tgt_points, sq_t, ones_t, jnp.zeros((B, M, 3), jnp.float32)], axis=-1
        ),
        (0, 2, 1),
    )

    out = pl.pallas_call(
        _chamfer_body,
        grid=(B,),
        in_specs=[
            pl.BlockSpec((1, N, 8), lambda b: (b, 0, 0)),
            pl.BlockSpec((1, 8, M), lambda b: (b, 0, 0)),
        ],
        out_specs=pl.BlockSpec((1, 1), lambda b: (0, 0), memory_space=pltpu.SMEM),
        out_shape=jax.ShapeDtypeStruct((1, 1), jnp.float32),
    )(src_aug, tgtT_aug)
    return out[0, 0]
